# baseline (speedup 1.0000x reference)
import jax
import jax.numpy as jnp
from jax.experimental import pallas as pl


def kernel(data, indices, updates):
    M = data.shape[0]
    B = updates.shape[0]
    idx = indices[:, 0]
    iota = jnp.arange(B, dtype=jnp.int32)
    tag = jnp.full((M,), -1, jnp.int32).at[idx].max(iota)
    keep = tag[idx] == iota
    safe_idx = jnp.where(keep, idx, M)
    return data.at[safe_idx].set(updates, mode="drop")


# speedup vs baseline: 1.6579x; 1.6579x over previous
import functools

import jax
import jax.numpy as jnp
from jax import lax
from jax.experimental import pallas as pl
from jax.experimental.pallas import tpu as pltpu
from jax.experimental.pallas import tpu_sc as plsc

_M = 1000000
_D = 64
_B = 16384
_NW = 32
_R = 31248
_EXTRA_BASE = _NW * _R
_TAG = _R + 16
_TAG_VREGS = _TAG // 16
_CBLK = 168
_NBLK = _R // _CBLK
_LCAP = _B + 80
_CHUNK = 64

_mesh = plsc.VectorSubcoreMesh(core_axis_name="c", subcore_axis_name="s")


@functools.partial(
    pl.kernel,
    out_type=jax.ShapeDtypeStruct((_M, _D), jnp.float32),
    mesh=_mesh,
    scratch_types=[
        pltpu.VMEM((_B,), jnp.int32),
        pltpu.VMEM((_TAG,), jnp.int32),
        pltpu.VMEM((_LCAP,), jnp.int32),
        pltpu.VMEM((_LCAP,), jnp.int32),
        pltpu.VMEM((_CBLK, _D), jnp.float32),
        pltpu.VMEM((_CBLK, _D), jnp.float32),
        pltpu.SemaphoreType.DMA,
        pltpu.SemaphoreType.DMA,
        pltpu.SemaphoreType.DMA,
        pltpu.SemaphoreType.DMA,
    ],
    compiler_params=pltpu.CompilerParams(needs_layout_passes=False),
)
def _scatter_nd(data_hbm, idx_hbm, upd_hbm, out_hbm,
                idx_v, tag_v, rowlist_v, wlist_v,
                cbuf0, cbuf1, si0, si1, so0, so1):
    g = lax.axis_index("s") * 2 + lax.axis_index("c")
    lo = g * _R
    e_lo = _EXTRA_BASE + g * 2
    lane = lax.iota(jnp.int32, 16)
    cbufs = (cbuf0, cbuf1)
    sin = (si0, si1)
    sout = (so0, so1)

    pltpu.sync_copy(idx_hbm, idx_v)

    @pl.loop(0, _TAG_VREGS)
    def _init(t):
        tag_v[pl.ds(t * 16, 16)] = jnp.full((16,), -1, jnp.int32)

    @pl.loop(0, _B // 16)
    def _build(t):
        v = idx_v[pl.ds(t * 16, 16)]
        ordv = lane + t * 16
        m1 = (v >= lo) & (v < lo + _R)
        plsc.store_scatter(tag_v, [v - lo], ordv, mask=m1)
        m2 = (v >= e_lo) & (v < e_lo + 2)
        plsc.store_scatter(tag_v, [v - e_lo + _R], ordv, mask=m2)

    def _compact(t, off):
        tv = tag_v[pl.ds(t * 16, 16)]
        m = tv >= 0
        mi = m.astype(jnp.int32)
        cnt = jnp.max(plsc.all_reduce_population_count(m))
        pos = off + plsc.cumsum(mi) - mi
        p = lane + t * 16
        rowv = jnp.where(p < _R, lo + p, e_lo + (p - _R))
        plsc.store_scatter(rowlist_v, [pos], rowv, mask=m)
        plsc.store_scatter(wlist_v, [pos], tv, mask=m)
        return off + cnt
    n = lax.fori_loop(0, _TAG_VREGS, _compact, jnp.int32(0))

    pltpu.make_async_copy(data_hbm.at[pl.ds(lo, _CBLK)], cbuf0, si0).start()

    @pl.loop(0, _NBLK, step=2)
    def _copy(k0):
        for b in (0, 1):
            k = k0 + b
            bb = cbufs[b]
            nb = cbufs[1 - b]

            @pl.when(k > 0)
            def _():
                pltpu.make_async_copy(
                    nb, out_hbm.at[pl.ds(lo, _CBLK)], sout[1 - b]).wait()

            @pl.when(k < _NBLK - 1)
            def _():
                pltpu.make_async_copy(
                    data_hbm.at[pl.ds(lo + (k + 1) * _CBLK, _CBLK)],
                    nb, sin[1 - b]).start()

            pltpu.make_async_copy(
                data_hbm.at[pl.ds(lo, _CBLK)], bb, sin[b]).wait()
            pltpu.make_async_copy(
                bb, out_hbm.at[pl.ds(lo + k * _CBLK, _CBLK)], sout[b]).start()

    pltpu.make_async_copy(cbuf1, out_hbm.at[pl.ds(lo, _CBLK)], so1).wait()

    pltpu.sync_copy(data_hbm.at[pl.ds(e_lo, 2)], cbuf0.at[pl.ds(0, 2)])
    pltpu.sync_copy(cbuf0.at[pl.ds(0, 2)], out_hbm.at[pl.ds(e_lo, 2)])

    @pl.when(n > 0)
    def _scatter_phase():
        f_r = rowlist_v[pl.ds(0, 16)]
        f_w = wlist_v[pl.ds(0, 16)]
        r0 = jnp.max(jnp.where(lane == 0, f_r, -1))
        w0 = jnp.max(jnp.where(lane == 0, f_w, -1))
        r0v = jnp.zeros((16,), jnp.int32) + r0
        w0v = jnp.zeros((16,), jnp.int32) + w0
        t0 = n // 16

        @pl.loop(t0, t0 + 5)
        def _fill(t):
            cur_r = rowlist_v[pl.ds(t * 16, 16)]
            cur_w = wlist_v[pl.ds(t * 16, 16)]
            mm = (lane + t * 16) >= n
            rowlist_v[pl.ds(t * 16, 16)] = jnp.where(mm, r0v, cur_r)
            wlist_v[pl.ds(t * 16, 16)] = jnp.where(mm, w0v, cur_w)

        nchunks = (n + _CHUNK - 1) // _CHUNK

        @pl.loop(0, nchunks)
        def _chunk(c):
            off = c * _CHUNK
            for t in range(_CHUNK // 16):
                rv = rowlist_v[pl.ds(off + t * 16, 16)]
                wv = wlist_v[pl.ds(off + t * 16, 16)]

                @pl.loop(0, 16)
                def _fire(j2, rv=rv, wv=wv):
                    r = jnp.max(jnp.where(lane == j2, rv, -1))
                    w = jnp.max(jnp.where(lane == j2, wv, -1))
                    pltpu.make_async_copy(
                        upd_hbm.at[pl.ds(w, 1)],
                        out_hbm.at[pl.ds(r, 1)],
                        so0,
                    ).start()

            @pl.loop(0, _CHUNK)
            def _drain(j):
                pltpu.make_async_copy(
                    upd_hbm.at[pl.ds(0, 1)],
                    out_hbm.at[pl.ds(0, 1)],
                    so0,
                ).wait()


def kernel(data, indices, updates):
    idx = indices.reshape(_B)
    return _scatter_nd(data, idx, updates)


# speedup vs baseline: 1.7220x; 1.0386x over previous
import functools

import jax
import jax.numpy as jnp
from jax import lax
from jax.experimental import pallas as pl
from jax.experimental.pallas import tpu as pltpu
from jax.experimental.pallas import tpu_sc as plsc

_M = 1000000
_D = 64
_B = 16384
_NW = 32
_R = 31248
_EXTRA_BASE = _NW * _R
_TAG = _R + 16
_TAG_VREGS = _TAG // 16
_LCAP = _B + 80
_CHUNK = 64
_CPB = 5000

_mesh = plsc.VectorSubcoreMesh(core_axis_name="c", subcore_axis_name="s")


@functools.partial(
    pl.kernel,
    out_type=(
        jax.ShapeDtypeStruct((_NW, _LCAP), jnp.int32),
        jax.ShapeDtypeStruct((_NW, _LCAP), jnp.int32),
        jax.ShapeDtypeStruct((_NW, 16), jnp.int32),
    ),
    mesh=_mesh,
    scratch_types=[
        pltpu.VMEM((_B,), jnp.int32),
        pltpu.VMEM((_TAG,), jnp.int32),
        pltpu.VMEM((_LCAP,), jnp.int32),
        pltpu.VMEM((_LCAP,), jnp.int32),
        pltpu.VMEM((16,), jnp.int32),
    ],
    compiler_params=pltpu.CompilerParams(needs_layout_passes=False),
)
def _sc_prep(idx_hbm, rowl_hbm, wl_hbm, cnt_hbm,
             idx_v, tag_v, rowlist_v, wlist_v, cnt_v):
    g = lax.axis_index("s") * 2 + lax.axis_index("c")
    lo = g * _R
    e_lo = _EXTRA_BASE + g * 2
    lane = lax.iota(jnp.int32, 16)

    pltpu.sync_copy(idx_hbm, idx_v)

    @pl.loop(0, _TAG_VREGS)
    def _init(t):
        tag_v[pl.ds(t * 16, 16)] = jnp.full((16,), -1, jnp.int32)

    @pl.loop(0, _B // 16)
    def _build(t):
        v = idx_v[pl.ds(t * 16, 16)]
        ordv = lane + t * 16
        m1 = (v >= lo) & (v < lo + _R)
        plsc.store_scatter(tag_v, [v - lo], ordv, mask=m1)
        m2 = (v >= e_lo) & (v < e_lo + 2)
        plsc.store_scatter(tag_v, [v - e_lo + _R], ordv, mask=m2)

    def _compact(t, off):
        tv = tag_v[pl.ds(t * 16, 16)]
        m = tv >= 0
        mi = m.astype(jnp.int32)
        cnt = jnp.max(plsc.all_reduce_population_count(m))
        pos = off + plsc.cumsum(mi) - mi
        p = lane + t * 16
        rowv = jnp.where(p < _R, lo + p, e_lo + (p - _R))
        plsc.store_scatter(rowlist_v, [pos], rowv, mask=m)
        plsc.store_scatter(wlist_v, [pos], tv, mask=m)
        return off + cnt
    n = lax.fori_loop(0, _TAG_VREGS, _compact, jnp.int32(0))

    cnt_v[pl.ds(0, 16)] = jnp.zeros((16,), jnp.int32) + n

    @pl.when(n > 0)
    def _pad():
        f_r = rowlist_v[pl.ds(0, 16)]
        f_w = wlist_v[pl.ds(0, 16)]
        r0 = jnp.max(jnp.where(lane == 0, f_r, -1))
        w0 = jnp.max(jnp.where(lane == 0, f_w, -1))
        r0v = jnp.zeros((16,), jnp.int32) + r0
        w0v = jnp.zeros((16,), jnp.int32) + w0
        t0 = n // 16

        @pl.loop(t0, t0 + 5)
        def _fill(t):
            cur_r = rowlist_v[pl.ds(t * 16, 16)]
            cur_w = wlist_v[pl.ds(t * 16, 16)]
            mm = (lane + t * 16) >= n
            rowlist_v[pl.ds(t * 16, 16)] = jnp.where(mm, r0v, cur_r)
            wlist_v[pl.ds(t * 16, 16)] = jnp.where(mm, w0v, cur_w)

    pltpu.sync_copy(rowlist_v, rowl_hbm.at[g])
    pltpu.sync_copy(wlist_v, wl_hbm.at[g])
    pltpu.sync_copy(cnt_v, cnt_hbm.at[g])


def _copy_body(x_ref, o_ref):
    o_ref[...] = x_ref[...]


_tc_copy = pl.pallas_call(
    _copy_body,
    out_shape=jax.ShapeDtypeStruct((_M, _D), jnp.float32),
    grid=(_M // _CPB,),
    in_specs=[pl.BlockSpec((_CPB, _D), lambda i: (i, 0))],
    out_specs=pl.BlockSpec((_CPB, _D), lambda i: (i, 0)),
)


@functools.partial(
    pl.kernel,
    out_type=(),
    mesh=_mesh,
    scratch_types=[
        pltpu.VMEM((_LCAP,), jnp.int32),
        pltpu.VMEM((_LCAP,), jnp.int32),
        pltpu.VMEM((16,), jnp.int32),
        pltpu.SemaphoreType.DMA,
    ],
    compiler_params=pltpu.CompilerParams(needs_layout_passes=False),
)
def _sc_apply(out_ref, rowl_hbm, wl_hbm, cnt_hbm, upd_hbm,
              rowlist_v, wlist_v, cnt_v, sd):
    g = lax.axis_index("s") * 2 + lax.axis_index("c")
    lane = lax.iota(jnp.int32, 16)

    pltpu.sync_copy(cnt_hbm.at[g], cnt_v)
    n = jnp.max(cnt_v[pl.ds(0, 16)])

    @pl.when(n > 0)
    def _apply():
        pltpu.sync_copy(rowl_hbm.at[g], rowlist_v)
        pltpu.sync_copy(wl_hbm.at[g], wlist_v)
        nchunks = (n + _CHUNK - 1) // _CHUNK

        @pl.loop(0, nchunks)
        def _chunk(c):
            off = c * _CHUNK
            for t in range(_CHUNK // 16):
                rv = rowlist_v[pl.ds(off + t * 16, 16)]
                wv = wlist_v[pl.ds(off + t * 16, 16)]

                @pl.loop(0, 16)
                def _fire(j2, rv=rv, wv=wv):
                    r = jnp.max(jnp.where(lane == j2, rv, -1))
                    w = jnp.max(jnp.where(lane == j2, wv, -1))
                    pltpu.make_async_copy(
                        upd_hbm.at[pl.ds(w, 1)],
                        out_ref.at[pl.ds(r, 1)],
                        sd,
                    ).start()

            @pl.loop(0, _CHUNK)
            def _drain(j):
                pltpu.make_async_copy(
                    upd_hbm.at[pl.ds(0, 1)],
                    out_ref.at[pl.ds(0, 1)],
                    sd,
                ).wait()


def kernel(data, indices, updates):
    idx = indices.reshape(_B)
    rowl, wl, cnt = _sc_prep(idx)
    out0 = _tc_copy(data)
    r = jax.new_ref(out0)
    _sc_apply(r, rowl, wl, cnt, updates)
    return r[...]
